# baseline (speedup 1.0000x reference)
import functools

import jax
import jax.numpy as jnp
from jax import lax
from jax.experimental import pallas as pl
from jax.experimental.pallas import tpu as pltpu
from jax.experimental.pallas import tpu_sc as plsc

D = 64
L = 200
NC = 2
NS = 16
NW = NC * NS
SUB = 100
NSUB = 8
C = SUB * NSUB


def _emb_body(word_hbm, pos_hbm, x2d_hbm, posidx_hbm, out_hbm,
              pos_sh, posidx_v, idx_v, rows_v, sem):
    n_rows = out_hbm.shape[0]
    per_w = n_rows // NW
    chunks = per_w // C
    sid = lax.axis_index("s")
    wid = lax.axis_index("c") * NS + sid

    @pl.when(sid == 0)
    def _():
        pltpu.sync_copy(pos_hbm, pos_sh)

    pltpu.sync_copy(posidx_hbm, posidx_v)
    plsc.subcore_barrier()

    def chunk(k, carry):
        row0 = pl.multiple_of(wid * per_w + k * C, C)
        xrow0 = pl.multiple_of(row0 // SUB, NSUB)
        pltpu.sync_copy(x2d_hbm.at[pl.ds(xrow0, NSUB)], idx_v)
        cps = [
            pltpu.async_copy(word_hbm.at[idx_v.at[j]],
                             rows_v.at[pl.ds(j * SUB, SUB)], sem)
            for j in range(NSUB)
        ]
        for cp in cps:
            cp.wait()
        for j in range(NSUB):
            pltpu.sync_copy(pos_sh.at[posidx_v.at[j]],
                            rows_v.at[pl.ds(j * SUB, SUB)], add=True)
        pltpu.sync_copy(rows_v, out_hbm.at[pl.ds(row0, C)])
        return carry

    lax.fori_loop(0, chunks, chunk, 0)


@functools.partial(jax.jit, static_argnames=("n_rows",))
def _emb(word_table, pos_table, x2d, posidx, n_rows):
    mesh = plsc.VectorSubcoreMesh(core_axis_name="c", subcore_axis_name="s",
                                  num_cores=NC, num_subcores=NS)
    return pl.kernel(
        _emb_body,
        out_type=jax.ShapeDtypeStruct((n_rows, D), jnp.float32),
        mesh=mesh,
        compiler_params=pltpu.CompilerParams(use_tc_tiling_on_sc=False),
        scratch_types=[
            pltpu.VMEM_SHARED((L, D), jnp.float32),
            pltpu.VMEM((NSUB, SUB), jnp.int32),
            pltpu.VMEM((NSUB, SUB), jnp.int32),
            pltpu.VMEM((C, D), jnp.float32),
            pltpu.SemaphoreType.DMA,
        ],
    )(word_table, pos_table, x2d, posidx)


def kernel(word_table, pos_table, x):
    Bx, Lx = x.shape
    n_rows = Bx * Lx
    x2d = x.reshape(n_rows // SUB, SUB).astype(jnp.int32)
    posidx = (jnp.arange(C, dtype=jnp.int32) % L).reshape(NSUB, SUB)
    out = _emb(word_table, pos_table, x2d, posidx, n_rows)
    return out.reshape(Bx, Lx, D)
